# baseline (speedup 1.0000x reference)
import functools

import jax
import jax.numpy as jnp
from jax import lax
from jax.experimental import pallas as pl
from jax.experimental.pallas import tpu as pltpu
from jax.experimental.pallas import tpu_sc as plsc

_EPS = 1e-05
_CAP = 256


def _attn_kernel(x_ref, wcat_ref, bcat_ref, wo_ref, bo_ref, g1_ref, bn1_ref,
                 wg_ref, bg_ref, x1_ref, i12_ref, w12_ref, qkv_scr, *, scale,
                 n_heads, head_dim, blk_m):
    step = pl.program_id(0)
    d_model = n_heads * head_dim

    @pl.when(step == 0)
    def _():
        x = x_ref[...]
        for j in range(3):
            qkv_scr[j] = jax.lax.dot_general(
                x, wcat_ref[j], (((1,), (1,)), ((), ())),
                preferred_element_type=jnp.float32,
            ) + bcat_ref[j]

    @pl.when(step > 0)
    def _():
        i = step - 1
        rows = pl.ds(i * blk_m, blk_m)
        o_parts = []
        for h in range(n_heads):
            sl = slice(h * head_dim, (h + 1) * head_dim)
            q = qkv_scr[0, rows, sl]
            k = qkv_scr[1, :, sl]
            v = qkv_scr[2, :, sl]
            s = jax.lax.dot_general(
                q, k, (((1,), (1,)), ((), ())),
                preferred_element_type=jnp.float32,
            ) * scale
            m = jnp.max(s, axis=-1, keepdims=True)
            p = jnp.exp(s - m)
            inv = 1.0 / jnp.sum(p, axis=-1, keepdims=True)
            pv = jnp.dot(
                p.astype(jnp.bfloat16), v.astype(jnp.bfloat16),
                preferred_element_type=jnp.float32)
            o_parts.append(pv * inv)
        o = jnp.concatenate(o_parts, axis=-1)
        attn = jax.lax.dot_general(
            o, wo_ref[...], (((1,), (1,)), ((), ())),
            preferred_element_type=jnp.float32,
        ) + bo_ref[...]
        z = x_ref[rows, :] + attn
        xn = _layer_norm(z, g1_ref[...], bn1_ref[...])
        x1_ref[...] = xn
        logits = jax.lax.dot_general(
            xn, wg_ref[...], (((1,), (1,)), ((), ())),
            preferred_element_type=jnp.float32,
        ) + bg_ref[...]
        mx = jnp.max(logits, axis=-1, keepdims=True)
        ex = jnp.exp(logits - mx)
        sc = ex / jnp.sum(ex, axis=-1, keepdims=True)
        ncols = sc.shape[-1]
        e_iota = jax.lax.broadcasted_iota(jnp.int32, sc.shape, 1)
        m1 = jnp.max(sc, axis=-1, keepdims=True)
        i1 = jnp.min(jnp.where(sc == m1, e_iota, ncols), axis=-1,
                     keepdims=True)
        sel1 = e_iota == i1
        s2 = jnp.where(sel1, -jnp.inf, sc)
        m2 = jnp.max(s2, axis=-1, keepdims=True)
        i2 = jnp.min(jnp.where(s2 == m2, e_iota, ncols), axis=-1,
                     keepdims=True)
        i12_ref[...] = jnp.concatenate([i1, i2], axis=-1)
        w12_ref[...] = jnp.concatenate([m1, m2], axis=-1)


def _layer_norm(z, g, b):
    m = jnp.mean(z, axis=-1, keepdims=True)
    c = z - m
    v = jnp.mean(c * c, axis=-1, keepdims=True)
    return c * jax.lax.rsqrt(v + _EPS) * g + b


def _route_kernel(i12_ref, slot_ref, be_ref, *, n_exp, cap, n_blk,
                  row_chunk):
    n_tok = i12_ref.shape[0]
    i12 = i12_ref[...]
    iota_e = lax.broadcasted_iota(jnp.int32, (n_tok, n_exp), 1)
    oh0 = (i12[:, 0:1] == iota_e).astype(jnp.bfloat16)
    oh1 = (i12[:, 1:2] == iota_e).astype(jnp.bfloat16)
    oh = jnp.concatenate([oh0, oh1], axis=1)
    col = lax.broadcasted_iota(jnp.int32, (row_chunk, n_tok), 1)
    cs = []
    for r0 in range(0, n_tok, row_chunk):
        row = lax.broadcasted_iota(jnp.int32, (row_chunk, n_tok), 0) + r0
        lt = (row >= col).astype(jnp.bfloat16)
        cs.append(lax.dot_general(
            lt, oh, (((1,), (0,)), ((), ())),
            preferred_element_type=jnp.float32))
    c = jnp.concatenate(cs, axis=0)
    c0 = c[:, :n_exp]
    c1 = c[:, n_exp:]
    tot0 = c[n_tok - 1:n_tok, :n_exp]
    tot1 = c[n_tok - 1:n_tok, n_exp:]
    counts = tot0 + tot1
    nblk = jnp.floor((counts + (cap - 1)) * (1.0 / cap))
    iu = lax.broadcasted_iota(jnp.int32, (n_exp, n_exp), 0)
    ju = lax.broadcasted_iota(jnp.int32, (n_exp, n_exp), 1)
    ut = (iu < ju).astype(jnp.float32)
    sb = lax.dot_general(nblk, ut, (((1,), (0,)), ((), ())),
                         preferred_element_type=jnp.float32)
    start = sb * float(cap)
    f0 = oh0.astype(jnp.float32)
    f1 = oh1.astype(jnp.float32)
    slot0 = jnp.sum(f0 * (start + c0 - 1.0), axis=1, keepdims=True)
    slot1 = jnp.sum(f1 * (start + tot0 + c1 - 1.0), axis=1, keepdims=True)
    slot_ref[...] = jnp.concatenate([slot0, slot1], axis=1).astype(jnp.int32)
    blkend = sb + nblk
    iota_b = lax.broadcasted_iota(jnp.int32, (1, n_blk), 1).astype(jnp.float32)
    lane_e = lax.broadcasted_iota(jnp.int32, (1, n_exp), 1)
    acc = jnp.zeros((1, n_blk), jnp.float32)
    for e in range(n_exp):
        end_e = jnp.sum(jnp.where(lane_e == e, blkend, 0.0))
        acc = acc + (iota_b >= end_e).astype(jnp.float32)
    bmap = jnp.minimum(acc, float(n_exp - 1))
    n_used = jnp.sum(nblk, axis=1, keepdims=True)
    be_ref[...] = jnp.concatenate([bmap, n_used], axis=1).astype(jnp.int32)


def _ffn_kernel(bex_ref, xg_ref, w1_ref, b1_ref, w2_ref, b2_ref, yg_ref, *,
                f_chunk, n_blk):
    b = pl.program_id(0)

    @pl.when(b < bex_ref[n_blk])
    def _():
        xb = xg_ref[...].astype(jnp.bfloat16)
        n_f = w1_ref.shape[1]
        y = b2_ref[0]
        for f0 in range(0, n_f, f_chunk):
            h = jax.lax.dot_general(
                xb, w1_ref[0, f0:f0 + f_chunk, :], (((1,), (1,)), ((), ())),
                preferred_element_type=jnp.float32,
            ) + b1_ref[0, :, f0:f0 + f_chunk]
            h = jnp.maximum(h, 0.0)
            y = y + jax.lax.dot_general(
                h.astype(jnp.bfloat16), w2_ref[0, :, f0:f0 + f_chunk],
                (((1,), (1,)), ((), ())),
                preferred_element_type=jnp.float32,
            )
        yg_ref[...] = y


def _final_kernel(x1_ref, y0_ref, y1_ref, w12_ref, g2_ref, bn2_ref, o_ref):
    w0 = w12_ref[:, 0:1]
    w1 = w12_ref[:, 1:2]
    z = x1_ref[...] + w0 * y0_ref[...] + w1 * y1_ref[...]
    o_ref[...] = _layer_norm(z, g2_ref[...], bn2_ref[...])


def _sc_dispatch(x1, s0, s1, n_slot):
    n_tok, d = x1.shape
    nw = 32
    chunk = n_tok // nw
    mesh = plsc.VectorSubcoreMesh(core_axis_name="c", subcore_axis_name="s")

    @functools.partial(
        pl.kernel, mesh=mesh,
        out_type=jax.ShapeDtypeStruct((n_slot, d), jnp.float32),
        scratch_types=[pltpu.VMEM((chunk,), jnp.int32),
                       pltpu.VMEM((chunk, d), jnp.float32),
                       pltpu.SemaphoreType.DMA],
    )
    def disp(x1_hbm, s0_hbm, s1_hbm, xg_hbm, idx_v, rows_v, sem):
        wid = lax.axis_index("s") * 2 + lax.axis_index("c")
        base = wid * chunk
        pltpu.sync_copy(x1_hbm.at[pl.ds(base, chunk)], rows_v)
        pltpu.sync_copy(s0_hbm.at[pl.ds(base, chunk)], idx_v)
        pltpu.async_copy(rows_v, xg_hbm.at[idx_v], sem).wait()
        pltpu.sync_copy(s1_hbm.at[pl.ds(base, chunk)], idx_v)
        pltpu.async_copy(rows_v, xg_hbm.at[idx_v], sem).wait()

    return disp(x1, s0, s1)


def _sc_combine(yg, s0, s1, n_tok):
    _, d = yg.shape
    nw = 32
    chunk = n_tok // nw
    mesh = plsc.VectorSubcoreMesh(core_axis_name="c", subcore_axis_name="s")

    @functools.partial(
        pl.kernel, mesh=mesh,
        out_type=(jax.ShapeDtypeStruct((n_tok, d), jnp.float32),
                  jax.ShapeDtypeStruct((n_tok, d), jnp.float32)),
        scratch_types=[pltpu.VMEM((chunk,), jnp.int32),
                       pltpu.VMEM((chunk, d), jnp.float32),
                       pltpu.SemaphoreType.DMA],
    )
    def comb(yg_hbm, s0_hbm, s1_hbm, y0_hbm, y1_hbm, idx_v, rows_v, sem):
        wid = lax.axis_index("s") * 2 + lax.axis_index("c")
        base = wid * chunk
        pltpu.sync_copy(s0_hbm.at[pl.ds(base, chunk)], idx_v)
        pltpu.async_copy(yg_hbm.at[idx_v], rows_v, sem).wait()
        pltpu.sync_copy(rows_v, y0_hbm.at[pl.ds(base, chunk)])
        pltpu.sync_copy(s1_hbm.at[pl.ds(base, chunk)], idx_v)
        pltpu.async_copy(yg_hbm.at[idx_v], rows_v, sem).wait()
        pltpu.sync_copy(rows_v, y1_hbm.at[pl.ds(base, chunk)])

    return comb(yg, s0, s1)


def kernel(src, Wq, bq, Wk, bk, Wv, bv, Wo, bo, Wg, bg, W1e, b1e, W2e, b2e,
           g1, bn1, g2, bn2):
    Bq, T, D = src.shape
    E, F, _ = W1e.shape
    Hh = 12
    HD = D // Hh
    x = src.reshape(T, D)

    wcat = jnp.stack([Wq, Wk, Wv], axis=0)
    bcat = jnp.stack([bq, bk, bv], axis=0).reshape(3, 1, D)
    BMA = 256
    scale = float(HD) ** -0.5
    blk_idx = lambda s: (jax.lax.max(s - 1, 0), 0)
    x1, i12, w12 = pl.pallas_call(
        functools.partial(_attn_kernel, scale=scale, n_heads=Hh, head_dim=HD,
                          blk_m=BMA),
        grid=(1 + T // BMA,),
        in_specs=[
            pl.BlockSpec((T, D), lambda s: (0, 0)),
            pl.BlockSpec((3, D, D), lambda s: (0, 0, 0)),
            pl.BlockSpec((3, 1, D), lambda s: (0, 0, 0)),
            pl.BlockSpec((D, D), lambda s: (0, 0)),
            pl.BlockSpec((1, D), lambda s: (0, 0)),
            pl.BlockSpec((1, D), lambda s: (0, 0)),
            pl.BlockSpec((1, D), lambda s: (0, 0)),
            pl.BlockSpec((E, D), lambda s: (0, 0)),
            pl.BlockSpec((1, E), lambda s: (0, 0)),
        ],
        out_specs=[
            pl.BlockSpec((BMA, D), blk_idx),
            pl.BlockSpec((BMA, 2), blk_idx),
            pl.BlockSpec((BMA, 2), blk_idx),
        ],
        out_shape=[
            jax.ShapeDtypeStruct((T, D), jnp.float32),
            jax.ShapeDtypeStruct((T, 2), jnp.int32),
            jax.ShapeDtypeStruct((T, 2), jnp.float32),
        ],
        scratch_shapes=[pltpu.VMEM((3, T, D), jnp.float32)],
        compiler_params=pltpu.CompilerParams(
            dimension_semantics=("arbitrary",)),
    )(x, wcat, bcat, Wo, bo.reshape(1, D), g1.reshape(1, D),
      bn1.reshape(1, D), Wg, bg.reshape(1, E))

    n_blk = 2 * T // _CAP + (E - 1)
    n_slot = n_blk * _CAP
    slot, be2 = pl.pallas_call(
        functools.partial(_route_kernel, n_exp=E, cap=_CAP, n_blk=n_blk,
                          row_chunk=512),
        grid=(1,),
        in_specs=[pl.BlockSpec((T, 2), lambda i: (0, 0))],
        out_specs=[
            pl.BlockSpec((T, 2), lambda i: (0, 0)),
            pl.BlockSpec((1, n_blk + 1), lambda i: (0, 0)),
        ],
        out_shape=[
            jax.ShapeDtypeStruct((T, 2), jnp.int32),
            jax.ShapeDtypeStruct((1, n_blk + 1), jnp.int32),
        ],
    )(i12)
    s0 = slot[:, 0]
    s1 = slot[:, 1]
    bex = be2.reshape(n_blk + 1)

    xg = _sc_dispatch(x1, s0, s1, n_slot)

    W1b = W1e.astype(jnp.bfloat16)
    W2b = W2e.astype(jnp.bfloat16)
    yg = pl.pallas_call(
        functools.partial(_ffn_kernel, f_chunk=512, n_blk=n_blk),
        grid_spec=pltpu.PrefetchScalarGridSpec(
            num_scalar_prefetch=1,
            grid=(n_blk,),
            in_specs=[
                pl.BlockSpec((_CAP, D), lambda b, be_r: (b, 0)),
                pl.BlockSpec((1, F, D), lambda b, be_r: (be_r[b], 0, 0)),
                pl.BlockSpec((1, 1, F), lambda b, be_r: (be_r[b], 0, 0)),
                pl.BlockSpec((1, D, F), lambda b, be_r: (be_r[b], 0, 0)),
                pl.BlockSpec((1, 1, D), lambda b, be_r: (be_r[b], 0, 0)),
            ],
            out_specs=pl.BlockSpec((_CAP, D), lambda b, be_r: (b, 0)),
        ),
        out_shape=jax.ShapeDtypeStruct((n_slot, D), jnp.float32),
        compiler_params=pltpu.CompilerParams(
            dimension_semantics=("arbitrary",)),
    )(bex, xg, W1b, b1e.reshape(E, 1, F), W2b, b2e.reshape(E, 1, D))

    y0, y1 = _sc_combine(yg, s0, s1, T)

    BM7 = 512
    out = pl.pallas_call(
        _final_kernel,
        grid=(T // BM7,),
        in_specs=[
            pl.BlockSpec((BM7, D), lambda i: (i, 0)),
            pl.BlockSpec((BM7, D), lambda i: (i, 0)),
            pl.BlockSpec((BM7, D), lambda i: (i, 0)),
            pl.BlockSpec((BM7, 2), lambda i: (i, 0)),
            pl.BlockSpec((1, D), lambda i: (0, 0)),
            pl.BlockSpec((1, D), lambda i: (0, 0)),
        ],
        out_specs=pl.BlockSpec((BM7, D), lambda i: (i, 0)),
        out_shape=jax.ShapeDtypeStruct((T, D), jnp.float32),
        compiler_params=pltpu.CompilerParams(
            dimension_semantics=("parallel",)),
    )(x1, y0, y1, w12, g2.reshape(1, D), bn2.reshape(1, D))

    return out.reshape(Bq, T, D)


# speedup vs baseline: 1.0177x; 1.0177x over previous
import functools

import jax
import jax.numpy as jnp
from jax import lax
from jax.experimental import pallas as pl
from jax.experimental.pallas import tpu as pltpu
from jax.experimental.pallas import tpu_sc as plsc

_EPS = 1e-05
_CAP = 256


def _qkv_kernel(x_ref, w_ref, b_ref, q_ref, kv_ref):
    j = pl.program_id(1)
    x = x_ref[...]
    w = w_ref[0]
    r = jax.lax.dot_general(
        x, w, (((1,), (1,)), ((), ())), preferred_element_type=jnp.float32
    ) + b_ref[0]

    @pl.when(j == 0)
    def _():
        q_ref[...] = r

    @pl.when(j > 0)
    def _():
        kv_ref[...] = r


def _attn_kernel(q_ref, kv_ref, src_ref, wo_ref, bo_ref, g1_ref, bn1_ref,
                 wg_ref, bg_ref, x1_ref, i12_ref, w12_ref, *, scale, n_heads,
                 head_dim):
    d_model = n_heads * head_dim
    o_parts = []
    for h in range(n_heads):
        sl = slice(h * head_dim, (h + 1) * head_dim)
        vsl = slice(d_model + h * head_dim, d_model + (h + 1) * head_dim)
        q = q_ref[:, sl]
        k = kv_ref[:, sl]
        v = kv_ref[:, vsl]
        s = jax.lax.dot_general(
            q, k, (((1,), (1,)), ((), ())), preferred_element_type=jnp.float32
        ) * scale
        m = jnp.max(s, axis=-1, keepdims=True)
        p = jnp.exp(s - m)
        inv = 1.0 / jnp.sum(p, axis=-1, keepdims=True)
        pv = jnp.dot(
            p.astype(jnp.bfloat16), v.astype(jnp.bfloat16),
            preferred_element_type=jnp.float32)
        o_parts.append(pv * inv)
    o = jnp.concatenate(o_parts, axis=-1)
    attn = jax.lax.dot_general(
        o, wo_ref[...], (((1,), (1,)), ((), ())),
        preferred_element_type=jnp.float32,
    ) + bo_ref[...]
    z = src_ref[...] + attn
    xn = _layer_norm(z, g1_ref[...], bn1_ref[...])
    x1_ref[...] = xn
    logits = jax.lax.dot_general(
        xn, wg_ref[...], (((1,), (1,)), ((), ())),
        preferred_element_type=jnp.float32,
    ) + bg_ref[...]
    mx = jnp.max(logits, axis=-1, keepdims=True)
    ex = jnp.exp(logits - mx)
    sc = ex / jnp.sum(ex, axis=-1, keepdims=True)
    ncols = sc.shape[-1]
    e_iota = jax.lax.broadcasted_iota(jnp.int32, sc.shape, 1)
    m1 = jnp.max(sc, axis=-1, keepdims=True)
    i1 = jnp.min(jnp.where(sc == m1, e_iota, ncols), axis=-1, keepdims=True)
    sel1 = e_iota == i1
    s2 = jnp.where(sel1, -jnp.inf, sc)
    m2 = jnp.max(s2, axis=-1, keepdims=True)
    i2 = jnp.min(jnp.where(s2 == m2, e_iota, ncols), axis=-1, keepdims=True)
    i12_ref[...] = jnp.concatenate([i1, i2], axis=-1)
    w12_ref[...] = jnp.concatenate([m1, m2], axis=-1)


def _layer_norm(z, g, b):
    m = jnp.mean(z, axis=-1, keepdims=True)
    c = z - m
    v = jnp.mean(c * c, axis=-1, keepdims=True)
    return c * jax.lax.rsqrt(v + _EPS) * g + b


def _route_kernel(i12_ref, slot_ref, be_ref, *, n_exp, cap, n_blk,
                  row_chunk):
    n_tok = i12_ref.shape[0]
    i12 = i12_ref[...]
    iota_e = lax.broadcasted_iota(jnp.int32, (n_tok, n_exp), 1)
    oh0 = (i12[:, 0:1] == iota_e).astype(jnp.bfloat16)
    oh1 = (i12[:, 1:2] == iota_e).astype(jnp.bfloat16)
    oh = jnp.concatenate([oh0, oh1], axis=1)
    col = lax.broadcasted_iota(jnp.int32, (row_chunk, n_tok), 1)
    cs = []
    for r0 in range(0, n_tok, row_chunk):
        row = lax.broadcasted_iota(jnp.int32, (row_chunk, n_tok), 0) + r0
        lt = (row >= col).astype(jnp.bfloat16)
        cs.append(lax.dot_general(
            lt, oh, (((1,), (0,)), ((), ())),
            preferred_element_type=jnp.float32))
    c = jnp.concatenate(cs, axis=0)
    c0 = c[:, :n_exp]
    c1 = c[:, n_exp:]
    tot0 = c[n_tok - 1:n_tok, :n_exp]
    tot1 = c[n_tok - 1:n_tok, n_exp:]
    counts = tot0 + tot1
    nblk = jnp.floor((counts + (cap - 1)) * (1.0 / cap))
    iu = lax.broadcasted_iota(jnp.int32, (n_exp, n_exp), 0)
    ju = lax.broadcasted_iota(jnp.int32, (n_exp, n_exp), 1)
    ut = (iu < ju).astype(jnp.float32)
    sb = lax.dot_general(nblk, ut, (((1,), (0,)), ((), ())),
                         preferred_element_type=jnp.float32)
    start = sb * float(cap)
    f0 = oh0.astype(jnp.float32)
    f1 = oh1.astype(jnp.float32)
    slot0 = jnp.sum(f0 * (start + c0 - 1.0), axis=1, keepdims=True)
    slot1 = jnp.sum(f1 * (start + tot0 + c1 - 1.0), axis=1, keepdims=True)
    slot_ref[...] = jnp.concatenate([slot0, slot1], axis=1).astype(jnp.int32)
    blkend = sb + nblk
    iota_b = lax.broadcasted_iota(jnp.int32, (1, n_blk), 1).astype(jnp.float32)
    lane_e = lax.broadcasted_iota(jnp.int32, (1, n_exp), 1)
    acc = jnp.zeros((1, n_blk), jnp.float32)
    for e in range(n_exp):
        end_e = jnp.sum(jnp.where(lane_e == e, blkend, 0.0))
        acc = acc + (iota_b >= end_e).astype(jnp.float32)
    bmap = jnp.minimum(acc, float(n_exp - 1))
    n_used = jnp.sum(nblk, axis=1, keepdims=True)
    be_ref[...] = jnp.concatenate([bmap, n_used], axis=1).astype(jnp.int32)


def _ffn_kernel(bex_ref, xg_ref, w1_ref, b1_ref, w2_ref, b2_ref, yg_ref, *,
                f_chunk, n_blk):
    b = pl.program_id(0)

    @pl.when(b < bex_ref[n_blk])
    def _():
        xb = xg_ref[...].astype(jnp.bfloat16)
        n_f = w1_ref.shape[1]
        y = b2_ref[0]
        for f0 in range(0, n_f, f_chunk):
            h = jax.lax.dot_general(
                xb, w1_ref[0, f0:f0 + f_chunk, :], (((1,), (1,)), ((), ())),
                preferred_element_type=jnp.float32,
            ) + b1_ref[0, :, f0:f0 + f_chunk]
            h = jnp.maximum(h, 0.0)
            y = y + jax.lax.dot_general(
                h.astype(jnp.bfloat16), w2_ref[0, :, f0:f0 + f_chunk],
                (((1,), (1,)), ((), ())),
                preferred_element_type=jnp.float32,
            )
        yg_ref[...] = y


def _final_kernel(x1_ref, y0_ref, y1_ref, w12_ref, g2_ref, bn2_ref, o_ref):
    w0 = w12_ref[:, 0:1]
    w1 = w12_ref[:, 1:2]
    z = x1_ref[...] + w0 * y0_ref[...] + w1 * y1_ref[...]
    o_ref[...] = _layer_norm(z, g2_ref[...], bn2_ref[...])


def _sc_dispatch(x1, s0, s1, n_slot):
    n_tok, d = x1.shape
    nw = 32
    chunk = n_tok // nw
    mesh = plsc.VectorSubcoreMesh(core_axis_name="c", subcore_axis_name="s")

    @functools.partial(
        pl.kernel, mesh=mesh,
        out_type=jax.ShapeDtypeStruct((n_slot, d), jnp.float32),
        scratch_types=[pltpu.VMEM((chunk,), jnp.int32),
                       pltpu.VMEM((chunk, d), jnp.float32),
                       pltpu.SemaphoreType.DMA],
    )
    def disp(x1_hbm, s0_hbm, s1_hbm, xg_hbm, idx_v, rows_v, sem):
        wid = lax.axis_index("s") * 2 + lax.axis_index("c")
        base = wid * chunk
        pltpu.sync_copy(x1_hbm.at[pl.ds(base, chunk)], rows_v)
        pltpu.sync_copy(s0_hbm.at[pl.ds(base, chunk)], idx_v)
        pltpu.async_copy(rows_v, xg_hbm.at[idx_v], sem).wait()
        pltpu.sync_copy(s1_hbm.at[pl.ds(base, chunk)], idx_v)
        pltpu.async_copy(rows_v, xg_hbm.at[idx_v], sem).wait()

    return disp(x1, s0, s1)


def _sc_combine(yg, s0, s1, n_tok):
    _, d = yg.shape
    nw = 32
    chunk = n_tok // nw
    mesh = plsc.VectorSubcoreMesh(core_axis_name="c", subcore_axis_name="s")

    @functools.partial(
        pl.kernel, mesh=mesh,
        out_type=(jax.ShapeDtypeStruct((n_tok, d), jnp.float32),
                  jax.ShapeDtypeStruct((n_tok, d), jnp.float32)),
        scratch_types=[pltpu.VMEM((chunk,), jnp.int32),
                       pltpu.VMEM((chunk, d), jnp.float32),
                       pltpu.SemaphoreType.DMA],
    )
    def comb(yg_hbm, s0_hbm, s1_hbm, y0_hbm, y1_hbm, idx_v, rows_v, sem):
        wid = lax.axis_index("s") * 2 + lax.axis_index("c")
        base = wid * chunk
        pltpu.sync_copy(s0_hbm.at[pl.ds(base, chunk)], idx_v)
        pltpu.async_copy(yg_hbm.at[idx_v], rows_v, sem).wait()
        pltpu.sync_copy(rows_v, y0_hbm.at[pl.ds(base, chunk)])
        pltpu.sync_copy(s1_hbm.at[pl.ds(base, chunk)], idx_v)
        pltpu.async_copy(yg_hbm.at[idx_v], rows_v, sem).wait()
        pltpu.sync_copy(rows_v, y1_hbm.at[pl.ds(base, chunk)])

    return comb(yg, s0, s1)


def kernel(src, Wq, bq, Wk, bk, Wv, bv, Wo, bo, Wg, bg, W1e, b1e, W2e, b2e,
           g1, bn1, g2, bn2):
    Bq, T, D = src.shape
    E, F, _ = W1e.shape
    Hh = 12
    HD = D // Hh
    x = src.reshape(T, D)

    wcat = jnp.stack([Wq, Wk, Wv], axis=0)
    bcat = jnp.stack([bq, bk, bv], axis=0).reshape(3, 1, D)
    BM1 = 512
    q, kv = pl.pallas_call(
        _qkv_kernel,
        grid=(T // BM1, 3),
        in_specs=[
            pl.BlockSpec((BM1, D), lambda i, j: (i, 0)),
            pl.BlockSpec((1, D, D), lambda i, j: (j, 0, 0)),
            pl.BlockSpec((1, 1, D), lambda i, j: (j, 0, 0)),
        ],
        out_specs=[
            pl.BlockSpec((BM1, D), lambda i, j: (i, 0)),
            pl.BlockSpec((BM1, D), lambda i, j: (i, jax.lax.max(j - 1, 0))),
        ],
        out_shape=[
            jax.ShapeDtypeStruct((T, D), jnp.float32),
            jax.ShapeDtypeStruct((T, 2 * D), jnp.float32),
        ],
        compiler_params=pltpu.CompilerParams(
            dimension_semantics=("parallel", "arbitrary")),
    )(x, wcat, bcat)

    BMA = 512
    scale = float(HD) ** -0.5
    x1, i12, w12 = pl.pallas_call(
        functools.partial(_attn_kernel, scale=scale, n_heads=Hh, head_dim=HD),
        grid=(T // BMA,),
        in_specs=[
            pl.BlockSpec((BMA, D), lambda i: (i, 0)),
            pl.BlockSpec((T, 2 * D), lambda i: (0, 0)),
            pl.BlockSpec((BMA, D), lambda i: (i, 0)),
            pl.BlockSpec((D, D), lambda i: (0, 0)),
            pl.BlockSpec((1, D), lambda i: (0, 0)),
            pl.BlockSpec((1, D), lambda i: (0, 0)),
            pl.BlockSpec((1, D), lambda i: (0, 0)),
            pl.BlockSpec((E, D), lambda i: (0, 0)),
            pl.BlockSpec((1, E), lambda i: (0, 0)),
        ],
        out_specs=[
            pl.BlockSpec((BMA, D), lambda i: (i, 0)),
            pl.BlockSpec((BMA, 2), lambda i: (i, 0)),
            pl.BlockSpec((BMA, 2), lambda i: (i, 0)),
        ],
        out_shape=[
            jax.ShapeDtypeStruct((T, D), jnp.float32),
            jax.ShapeDtypeStruct((T, 2), jnp.int32),
            jax.ShapeDtypeStruct((T, 2), jnp.float32),
        ],
        compiler_params=pltpu.CompilerParams(
            dimension_semantics=("arbitrary",)),
    )(q, kv, x, Wo, bo.reshape(1, D), g1.reshape(1, D), bn1.reshape(1, D),
      Wg, bg.reshape(1, E))

    n_blk = 2 * T // _CAP + (E - 1)
    n_slot = n_blk * _CAP
    slot, be2 = pl.pallas_call(
        functools.partial(_route_kernel, n_exp=E, cap=_CAP, n_blk=n_blk,
                          row_chunk=512),
        grid=(1,),
        in_specs=[pl.BlockSpec((T, 2), lambda i: (0, 0))],
        out_specs=[
            pl.BlockSpec((T, 2), lambda i: (0, 0)),
            pl.BlockSpec((1, n_blk + 1), lambda i: (0, 0)),
        ],
        out_shape=[
            jax.ShapeDtypeStruct((T, 2), jnp.int32),
            jax.ShapeDtypeStruct((1, n_blk + 1), jnp.int32),
        ],
    )(i12)
    s0 = slot[:, 0]
    s1 = slot[:, 1]
    bex = be2.reshape(n_blk + 1)

    xg = _sc_dispatch(x1, s0, s1, n_slot)

    W1b = W1e.astype(jnp.bfloat16)
    W2b = W2e.astype(jnp.bfloat16)
    yg = pl.pallas_call(
        functools.partial(_ffn_kernel, f_chunk=512, n_blk=n_blk),
        grid_spec=pltpu.PrefetchScalarGridSpec(
            num_scalar_prefetch=1,
            grid=(n_blk,),
            in_specs=[
                pl.BlockSpec((_CAP, D), lambda b, be_r: (b, 0)),
                pl.BlockSpec((1, F, D), lambda b, be_r: (be_r[b], 0, 0)),
                pl.BlockSpec((1, 1, F), lambda b, be_r: (be_r[b], 0, 0)),
                pl.BlockSpec((1, D, F), lambda b, be_r: (be_r[b], 0, 0)),
                pl.BlockSpec((1, 1, D), lambda b, be_r: (be_r[b], 0, 0)),
            ],
            out_specs=pl.BlockSpec((_CAP, D), lambda b, be_r: (b, 0)),
        ),
        out_shape=jax.ShapeDtypeStruct((n_slot, D), jnp.float32),
        compiler_params=pltpu.CompilerParams(
            dimension_semantics=("arbitrary",)),
    )(bex, xg, W1b, b1e.reshape(E, 1, F), W2b, b2e.reshape(E, 1, D))

    y0, y1 = _sc_combine(yg, s0, s1, T)

    BM7 = 512
    out = pl.pallas_call(
        _final_kernel,
        grid=(T // BM7,),
        in_specs=[
            pl.BlockSpec((BM7, D), lambda i: (i, 0)),
            pl.BlockSpec((BM7, D), lambda i: (i, 0)),
            pl.BlockSpec((BM7, D), lambda i: (i, 0)),
            pl.BlockSpec((BM7, 2), lambda i: (i, 0)),
            pl.BlockSpec((1, D), lambda i: (0, 0)),
            pl.BlockSpec((1, D), lambda i: (0, 0)),
        ],
        out_specs=pl.BlockSpec((BM7, D), lambda i: (i, 0)),
        out_shape=jax.ShapeDtypeStruct((T, D), jnp.float32),
        compiler_params=pltpu.CompilerParams(
            dimension_semantics=("parallel",)),
    )(x1, y0, y1, w12, g2.reshape(1, D), bn2.reshape(1, D))

    return out.reshape(Bq, T, D)


# speedup vs baseline: 1.0856x; 1.0666x over previous
import functools

import jax
import jax.numpy as jnp
from jax import lax
from jax.experimental import pallas as pl
from jax.experimental.pallas import tpu as pltpu
from jax.experimental.pallas import tpu_sc as plsc

_EPS = 1e-05
_CAP = 256


def _qkv_kernel(x_ref, w_ref, b_ref, q_ref, kv_ref):
    j = pl.program_id(1)
    x = x_ref[...]
    w = w_ref[0]
    r = jax.lax.dot_general(
        x, w, (((1,), (1,)), ((), ())), preferred_element_type=jnp.float32
    ) + b_ref[0]

    @pl.when(j == 0)
    def _():
        q_ref[...] = r

    @pl.when(j > 0)
    def _():
        kv_ref[...] = r


def _attn_kernel(q_ref, kv_ref, src_ref, wo_ref, bo_ref, g1_ref, bn1_ref,
                 wg_ref, bg_ref, x1_ref, i12_ref, w12_ref, *, scale, n_heads,
                 head_dim):
    d_model = n_heads * head_dim
    o_parts = []
    for h in range(n_heads):
        sl = slice(h * head_dim, (h + 1) * head_dim)
        vsl = slice(d_model + h * head_dim, d_model + (h + 1) * head_dim)
        q = q_ref[:, sl]
        k = kv_ref[:, sl]
        v = kv_ref[:, vsl]
        s = jax.lax.dot_general(
            q, k, (((1,), (1,)), ((), ())), preferred_element_type=jnp.float32
        ) * scale
        m = jnp.max(s, axis=-1, keepdims=True)
        p = jnp.exp(s - m)
        inv = 1.0 / jnp.sum(p, axis=-1, keepdims=True)
        pv = jnp.dot(
            p.astype(jnp.bfloat16), v.astype(jnp.bfloat16),
            preferred_element_type=jnp.float32)
        o_parts.append(pv * inv)
    o = jnp.concatenate(o_parts, axis=-1)
    attn = jax.lax.dot_general(
        o, wo_ref[...], (((1,), (1,)), ((), ())),
        preferred_element_type=jnp.float32,
    ) + bo_ref[...]
    z = src_ref[...] + attn
    xn = _layer_norm(z, g1_ref[...], bn1_ref[...])
    x1_ref[...] = xn
    logits = jax.lax.dot_general(
        xn, wg_ref[...], (((1,), (1,)), ((), ())),
        preferred_element_type=jnp.float32,
    ) + bg_ref[...]
    mx = jnp.max(logits, axis=-1, keepdims=True)
    ex = jnp.exp(logits - mx)
    sc = ex / jnp.sum(ex, axis=-1, keepdims=True)
    ncols = sc.shape[-1]
    e_iota = jax.lax.broadcasted_iota(jnp.int32, sc.shape, 1)
    m1 = jnp.max(sc, axis=-1, keepdims=True)
    i1 = jnp.min(jnp.where(sc == m1, e_iota, ncols), axis=-1, keepdims=True)
    sel1 = e_iota == i1
    s2 = jnp.where(sel1, -jnp.inf, sc)
    m2 = jnp.max(s2, axis=-1, keepdims=True)
    i2 = jnp.min(jnp.where(s2 == m2, e_iota, ncols), axis=-1, keepdims=True)
    i12_ref[...] = jnp.concatenate([i1, i2], axis=-1)
    w12_ref[...] = jnp.concatenate([m1, m2], axis=-1)


def _layer_norm(z, g, b):
    m = jnp.mean(z, axis=-1, keepdims=True)
    c = z - m
    v = jnp.mean(c * c, axis=-1, keepdims=True)
    return c * jax.lax.rsqrt(v + _EPS) * g + b


def _route_kernel(i12_ref, slot_ref, be_ref, *, n_exp, cap, n_blk,
                  row_chunk):
    n_tok = i12_ref.shape[0]
    i12 = i12_ref[...]
    iota_e = lax.broadcasted_iota(jnp.int32, (n_tok, n_exp), 1)
    oh0 = (i12[:, 0:1] == iota_e).astype(jnp.bfloat16)
    oh1 = (i12[:, 1:2] == iota_e).astype(jnp.bfloat16)
    oh = jnp.concatenate([oh0, oh1], axis=1)
    col = lax.broadcasted_iota(jnp.int32, (row_chunk, n_tok), 1)
    cs = []
    for r0 in range(0, n_tok, row_chunk):
        row = lax.broadcasted_iota(jnp.int32, (row_chunk, n_tok), 0) + r0
        lt = (row >= col).astype(jnp.bfloat16)
        cs.append(lax.dot_general(
            lt, oh, (((1,), (0,)), ((), ())),
            preferred_element_type=jnp.float32))
    c = jnp.concatenate(cs, axis=0)
    c0 = c[:, :n_exp]
    c1 = c[:, n_exp:]
    tot0 = c[n_tok - 1:n_tok, :n_exp]
    tot1 = c[n_tok - 1:n_tok, n_exp:]
    counts = tot0 + tot1
    nblk = jnp.floor((counts + (cap - 1)) * (1.0 / cap))
    iu = lax.broadcasted_iota(jnp.int32, (n_exp, n_exp), 0)
    ju = lax.broadcasted_iota(jnp.int32, (n_exp, n_exp), 1)
    ut = (iu < ju).astype(jnp.float32)
    sb = lax.dot_general(nblk, ut, (((1,), (0,)), ((), ())),
                         preferred_element_type=jnp.float32)
    start = sb * float(cap)
    f0 = oh0.astype(jnp.float32)
    f1 = oh1.astype(jnp.float32)
    slot0 = jnp.sum(f0 * (start + c0 - 1.0), axis=1, keepdims=True)
    slot1 = jnp.sum(f1 * (start + tot0 + c1 - 1.0), axis=1, keepdims=True)
    slot_ref[...] = jnp.concatenate([slot0, slot1], axis=1).astype(jnp.int32)
    blkend = sb + nblk
    iota_b = lax.broadcasted_iota(jnp.int32, (1, n_blk), 1).astype(jnp.float32)
    lane_e = lax.broadcasted_iota(jnp.int32, (1, n_exp), 1)
    acc = jnp.zeros((1, n_blk), jnp.float32)
    for e in range(n_exp):
        end_e = jnp.sum(jnp.where(lane_e == e, blkend, 0.0))
        acc = acc + (iota_b >= end_e).astype(jnp.float32)
    bmap = jnp.minimum(acc, float(n_exp - 1))
    n_used = jnp.sum(nblk, axis=1, keepdims=True)
    be_ref[...] = jnp.concatenate([bmap, n_used], axis=1).astype(jnp.int32)


def _ffn_kernel(bex_ref, xg_ref, w1_ref, b1_ref, w2_ref, b2_ref, yg_ref, *,
                f_chunk, n_blk):
    b = pl.program_id(0)

    @pl.when(b < bex_ref[n_blk])
    def _():
        xb = xg_ref[...].astype(jnp.bfloat16)
        n_f = w1_ref.shape[1]
        y = b2_ref[0]
        for f0 in range(0, n_f, f_chunk):
            w1c = w1_ref[0, f0:f0 + f_chunk, :].astype(jnp.bfloat16)
            h = jax.lax.dot_general(
                xb, w1c, (((1,), (1,)), ((), ())),
                preferred_element_type=jnp.float32,
            ) + b1_ref[0, :, f0:f0 + f_chunk]
            h = jnp.maximum(h, 0.0)
            w2c = w2_ref[0, :, f0:f0 + f_chunk].astype(jnp.bfloat16)
            y = y + jax.lax.dot_general(
                h.astype(jnp.bfloat16), w2c, (((1,), (1,)), ((), ())),
                preferred_element_type=jnp.float32,
            )
        yg_ref[...] = y


def _final_kernel(x1_ref, y0_ref, y1_ref, w12_ref, g2_ref, bn2_ref, o_ref):
    w0 = w12_ref[:, 0:1]
    w1 = w12_ref[:, 1:2]
    z = x1_ref[...] + w0 * y0_ref[...] + w1 * y1_ref[...]
    o_ref[...] = _layer_norm(z, g2_ref[...], bn2_ref[...])


def _sc_dispatch(x1, s0, s1, n_slot):
    n_tok, d = x1.shape
    nw = 32
    chunk = n_tok // nw
    mesh = plsc.VectorSubcoreMesh(core_axis_name="c", subcore_axis_name="s")

    @functools.partial(
        pl.kernel, mesh=mesh,
        out_type=jax.ShapeDtypeStruct((n_slot, d), jnp.float32),
        scratch_types=[pltpu.VMEM((chunk,), jnp.int32),
                       pltpu.VMEM((chunk, d), jnp.float32),
                       pltpu.SemaphoreType.DMA],
    )
    def disp(x1_hbm, s0_hbm, s1_hbm, xg_hbm, idx_v, rows_v, sem):
        wid = lax.axis_index("s") * 2 + lax.axis_index("c")
        base = wid * chunk
        pltpu.sync_copy(x1_hbm.at[pl.ds(base, chunk)], rows_v)
        pltpu.sync_copy(s0_hbm.at[pl.ds(base, chunk)], idx_v)
        pltpu.async_copy(rows_v, xg_hbm.at[idx_v], sem).wait()
        pltpu.sync_copy(s1_hbm.at[pl.ds(base, chunk)], idx_v)
        pltpu.async_copy(rows_v, xg_hbm.at[idx_v], sem).wait()

    return disp(x1, s0, s1)


def _sc_combine(yg, s0, s1, n_tok):
    _, d = yg.shape
    nw = 32
    chunk = n_tok // nw
    mesh = plsc.VectorSubcoreMesh(core_axis_name="c", subcore_axis_name="s")

    @functools.partial(
        pl.kernel, mesh=mesh,
        out_type=(jax.ShapeDtypeStruct((n_tok, d), jnp.float32),
                  jax.ShapeDtypeStruct((n_tok, d), jnp.float32)),
        scratch_types=[pltpu.VMEM((chunk,), jnp.int32),
                       pltpu.VMEM((chunk, d), jnp.float32),
                       pltpu.SemaphoreType.DMA],
    )
    def comb(yg_hbm, s0_hbm, s1_hbm, y0_hbm, y1_hbm, idx_v, rows_v, sem):
        wid = lax.axis_index("s") * 2 + lax.axis_index("c")
        base = wid * chunk
        pltpu.sync_copy(s0_hbm.at[pl.ds(base, chunk)], idx_v)
        pltpu.async_copy(yg_hbm.at[idx_v], rows_v, sem).wait()
        pltpu.sync_copy(rows_v, y0_hbm.at[pl.ds(base, chunk)])
        pltpu.sync_copy(s1_hbm.at[pl.ds(base, chunk)], idx_v)
        pltpu.async_copy(yg_hbm.at[idx_v], rows_v, sem).wait()
        pltpu.sync_copy(rows_v, y1_hbm.at[pl.ds(base, chunk)])

    return comb(yg, s0, s1)


def kernel(src, Wq, bq, Wk, bk, Wv, bv, Wo, bo, Wg, bg, W1e, b1e, W2e, b2e,
           g1, bn1, g2, bn2):
    Bq, T, D = src.shape
    E, F, _ = W1e.shape
    Hh = 12
    HD = D // Hh
    x = src.reshape(T, D)

    wcat = jnp.stack([Wq, Wk, Wv], axis=0)
    bcat = jnp.stack([bq, bk, bv], axis=0).reshape(3, 1, D)
    BM1 = 512
    q, kv = pl.pallas_call(
        _qkv_kernel,
        grid=(T // BM1, 3),
        in_specs=[
            pl.BlockSpec((BM1, D), lambda i, j: (i, 0)),
            pl.BlockSpec((1, D, D), lambda i, j: (j, 0, 0)),
            pl.BlockSpec((1, 1, D), lambda i, j: (j, 0, 0)),
        ],
        out_specs=[
            pl.BlockSpec((BM1, D), lambda i, j: (i, 0)),
            pl.BlockSpec((BM1, D), lambda i, j: (i, jax.lax.max(j - 1, 0))),
        ],
        out_shape=[
            jax.ShapeDtypeStruct((T, D), jnp.float32),
            jax.ShapeDtypeStruct((T, 2 * D), jnp.float32),
        ],
        compiler_params=pltpu.CompilerParams(
            dimension_semantics=("parallel", "arbitrary")),
    )(x, wcat, bcat)

    BMA = 512
    scale = float(HD) ** -0.5
    x1, i12, w12 = pl.pallas_call(
        functools.partial(_attn_kernel, scale=scale, n_heads=Hh, head_dim=HD),
        grid=(T // BMA,),
        in_specs=[
            pl.BlockSpec((BMA, D), lambda i: (i, 0)),
            pl.BlockSpec((T, 2 * D), lambda i: (0, 0)),
            pl.BlockSpec((BMA, D), lambda i: (i, 0)),
            pl.BlockSpec((D, D), lambda i: (0, 0)),
            pl.BlockSpec((1, D), lambda i: (0, 0)),
            pl.BlockSpec((1, D), lambda i: (0, 0)),
            pl.BlockSpec((1, D), lambda i: (0, 0)),
            pl.BlockSpec((E, D), lambda i: (0, 0)),
            pl.BlockSpec((1, E), lambda i: (0, 0)),
        ],
        out_specs=[
            pl.BlockSpec((BMA, D), lambda i: (i, 0)),
            pl.BlockSpec((BMA, 2), lambda i: (i, 0)),
            pl.BlockSpec((BMA, 2), lambda i: (i, 0)),
        ],
        out_shape=[
            jax.ShapeDtypeStruct((T, D), jnp.float32),
            jax.ShapeDtypeStruct((T, 2), jnp.int32),
            jax.ShapeDtypeStruct((T, 2), jnp.float32),
        ],
        compiler_params=pltpu.CompilerParams(
            dimension_semantics=("arbitrary",)),
    )(q, kv, x, Wo, bo.reshape(1, D), g1.reshape(1, D), bn1.reshape(1, D),
      Wg, bg.reshape(1, E))

    n_blk = 2 * T // _CAP + (E - 1)
    n_slot = n_blk * _CAP
    slot, be2 = pl.pallas_call(
        functools.partial(_route_kernel, n_exp=E, cap=_CAP, n_blk=n_blk,
                          row_chunk=512),
        grid=(1,),
        in_specs=[pl.BlockSpec((T, 2), lambda i: (0, 0))],
        out_specs=[
            pl.BlockSpec((T, 2), lambda i: (0, 0)),
            pl.BlockSpec((1, n_blk + 1), lambda i: (0, 0)),
        ],
        out_shape=[
            jax.ShapeDtypeStruct((T, 2), jnp.int32),
            jax.ShapeDtypeStruct((1, n_blk + 1), jnp.int32),
        ],
    )(i12)
    s0 = slot[:, 0]
    s1 = slot[:, 1]
    bex = be2.reshape(n_blk + 1)

    xg = _sc_dispatch(x1, s0, s1, n_slot)

    yg = pl.pallas_call(
        functools.partial(_ffn_kernel, f_chunk=512, n_blk=n_blk),
        grid_spec=pltpu.PrefetchScalarGridSpec(
            num_scalar_prefetch=1,
            grid=(n_blk,),
            in_specs=[
                pl.BlockSpec((_CAP, D), lambda b, be_r: (b, 0)),
                pl.BlockSpec((1, F, D), lambda b, be_r: (be_r[b], 0, 0)),
                pl.BlockSpec((1, 1, F), lambda b, be_r: (be_r[b], 0, 0)),
                pl.BlockSpec((1, D, F), lambda b, be_r: (be_r[b], 0, 0)),
                pl.BlockSpec((1, 1, D), lambda b, be_r: (be_r[b], 0, 0)),
            ],
            out_specs=pl.BlockSpec((_CAP, D), lambda b, be_r: (b, 0)),
        ),
        out_shape=jax.ShapeDtypeStruct((n_slot, D), jnp.float32),
        compiler_params=pltpu.CompilerParams(
            dimension_semantics=("arbitrary",)),
    )(bex, xg, W1e, b1e.reshape(E, 1, F), W2e, b2e.reshape(E, 1, D))

    y0, y1 = _sc_combine(yg, s0, s1, T)

    BM7 = 512
    out = pl.pallas_call(
        _final_kernel,
        grid=(T // BM7,),
        in_specs=[
            pl.BlockSpec((BM7, D), lambda i: (i, 0)),
            pl.BlockSpec((BM7, D), lambda i: (i, 0)),
            pl.BlockSpec((BM7, D), lambda i: (i, 0)),
            pl.BlockSpec((BM7, 2), lambda i: (i, 0)),
            pl.BlockSpec((1, D), lambda i: (0, 0)),
            pl.BlockSpec((1, D), lambda i: (0, 0)),
        ],
        out_specs=pl.BlockSpec((BM7, D), lambda i: (i, 0)),
        out_shape=jax.ShapeDtypeStruct((T, D), jnp.float32),
        compiler_params=pltpu.CompilerParams(
            dimension_semantics=("parallel",)),
    )(x1, y0, y1, w12, g2.reshape(1, D), bn2.reshape(1, D))

    return out.reshape(Bq, T, D)


# speedup vs baseline: 1.0919x; 1.0059x over previous
import functools

import jax
import jax.numpy as jnp
from jax import lax
from jax.experimental import pallas as pl
from jax.experimental.pallas import tpu as pltpu
from jax.experimental.pallas import tpu_sc as plsc

_EPS = 1e-05
_CAP = 256


def _qkv_kernel(x_ref, w_ref, b_ref, q_ref, kv_ref):
    j = pl.program_id(1)
    x = x_ref[...]
    w = w_ref[0]
    r = jax.lax.dot_general(
        x, w, (((1,), (1,)), ((), ())), preferred_element_type=jnp.float32
    ) + b_ref[0]

    @pl.when(j == 0)
    def _():
        q_ref[...] = r

    @pl.when(j > 0)
    def _():
        kv_ref[...] = r


def _attn_kernel(q_ref, kv_ref, src_ref, wo_ref, bo_ref, g1_ref, bn1_ref,
                 wg_ref, bg_ref, x1_ref, i12_ref, w12_ref, *, scale, n_heads,
                 head_dim):
    d_model = n_heads * head_dim
    o_parts = []
    for h in range(n_heads):
        sl = slice(h * head_dim, (h + 1) * head_dim)
        vsl = slice(d_model + h * head_dim, d_model + (h + 1) * head_dim)
        q = q_ref[:, sl]
        k = kv_ref[:, sl]
        v = kv_ref[:, vsl]
        s = jax.lax.dot_general(
            q, k, (((1,), (1,)), ((), ())), preferred_element_type=jnp.float32
        ) * scale
        m = jnp.max(s, axis=-1, keepdims=True)
        p = jnp.exp(s - m)
        inv = 1.0 / jnp.sum(p, axis=-1, keepdims=True)
        pv = jnp.dot(
            p.astype(jnp.bfloat16), v.astype(jnp.bfloat16),
            preferred_element_type=jnp.float32)
        o_parts.append(pv * inv)
    o = jnp.concatenate(o_parts, axis=-1)
    attn = jax.lax.dot_general(
        o, wo_ref[...], (((1,), (1,)), ((), ())),
        preferred_element_type=jnp.float32,
    ) + bo_ref[...]
    z = src_ref[...] + attn
    xn = _layer_norm(z, g1_ref[...], bn1_ref[...])
    x1_ref[...] = xn
    logits = jax.lax.dot_general(
        xn, wg_ref[...], (((1,), (1,)), ((), ())),
        preferred_element_type=jnp.float32,
    ) + bg_ref[...]
    mx = jnp.max(logits, axis=-1, keepdims=True)
    ex = jnp.exp(logits - mx)
    sc = ex / jnp.sum(ex, axis=-1, keepdims=True)
    ncols = sc.shape[-1]
    e_iota = jax.lax.broadcasted_iota(jnp.int32, sc.shape, 1)
    m1 = jnp.max(sc, axis=-1, keepdims=True)
    i1 = jnp.min(jnp.where(sc == m1, e_iota, ncols), axis=-1, keepdims=True)
    sel1 = e_iota == i1
    s2 = jnp.where(sel1, -jnp.inf, sc)
    m2 = jnp.max(s2, axis=-1, keepdims=True)
    i2 = jnp.min(jnp.where(s2 == m2, e_iota, ncols), axis=-1, keepdims=True)
    i12_ref[...] = jnp.concatenate([i1, i2], axis=-1)
    w12_ref[...] = jnp.concatenate([m1, m2], axis=-1)


def _layer_norm(z, g, b):
    m = jnp.mean(z, axis=-1, keepdims=True)
    c = z - m
    v = jnp.mean(c * c, axis=-1, keepdims=True)
    return c * jax.lax.rsqrt(v + _EPS) * g + b


def _route_kernel(i12_ref, slot_ref, be_ref, *, n_exp, cap, n_blk,
                  row_chunk):
    n_tok = i12_ref.shape[0]
    i12 = i12_ref[...]
    iota_e = lax.broadcasted_iota(jnp.int32, (n_tok, n_exp), 1)
    oh0 = (i12[:, 0:1] == iota_e).astype(jnp.bfloat16)
    oh1 = (i12[:, 1:2] == iota_e).astype(jnp.bfloat16)
    oh = jnp.concatenate([oh0, oh1], axis=1)
    col = lax.broadcasted_iota(jnp.int32, (row_chunk, n_tok), 1)
    cs = []
    for r0 in range(0, n_tok, row_chunk):
        row = lax.broadcasted_iota(jnp.int32, (row_chunk, n_tok), 0) + r0
        lt = (row >= col).astype(jnp.bfloat16)
        cs.append(lax.dot_general(
            lt, oh, (((1,), (0,)), ((), ())),
            preferred_element_type=jnp.float32))
    c = jnp.concatenate(cs, axis=0)
    c0 = c[:, :n_exp]
    c1 = c[:, n_exp:]
    tot0 = c[n_tok - 1:n_tok, :n_exp]
    tot1 = c[n_tok - 1:n_tok, n_exp:]
    counts = tot0 + tot1
    nblk = jnp.floor((counts + (cap - 1)) * (1.0 / cap))
    iu = lax.broadcasted_iota(jnp.int32, (n_exp, n_exp), 0)
    ju = lax.broadcasted_iota(jnp.int32, (n_exp, n_exp), 1)
    ut = (iu < ju).astype(jnp.float32)
    sb = lax.dot_general(nblk, ut, (((1,), (0,)), ((), ())),
                         preferred_element_type=jnp.float32)
    start = sb * float(cap)
    f0 = oh0.astype(jnp.float32)
    f1 = oh1.astype(jnp.float32)
    slot0 = jnp.sum(f0 * (start + c0 - 1.0), axis=1, keepdims=True)
    slot1 = jnp.sum(f1 * (start + tot0 + c1 - 1.0), axis=1, keepdims=True)
    slot_ref[...] = jnp.concatenate([slot0, slot1], axis=1).astype(jnp.int32)
    blkend = sb + nblk
    iota_b = lax.broadcasted_iota(jnp.int32, (1, n_blk), 1).astype(jnp.float32)
    lane_e = lax.broadcasted_iota(jnp.int32, (1, n_exp), 1)
    acc = jnp.zeros((1, n_blk), jnp.float32)
    for e in range(n_exp):
        end_e = jnp.sum(jnp.where(lane_e == e, blkend, 0.0))
        acc = acc + (iota_b >= end_e).astype(jnp.float32)
    bmap = jnp.minimum(acc, float(n_exp - 1))
    n_used = jnp.sum(nblk, axis=1, keepdims=True)
    be_ref[...] = jnp.concatenate([bmap, n_used], axis=1).astype(jnp.int32)


def _ffn_kernel(bex_ref, xg_ref, w1_ref, b1_ref, w2_ref, b2_ref, yg_ref, *,
                f_chunk, n_blk):
    b = pl.program_id(0)

    @pl.when(b < bex_ref[n_blk])
    def _():
        xb = xg_ref[...].astype(jnp.bfloat16)
        n_f = w1_ref.shape[1]
        y = b2_ref[0]
        for f0 in range(0, n_f, f_chunk):
            w1c = w1_ref[0, f0:f0 + f_chunk, :].astype(jnp.bfloat16)
            h = jax.lax.dot_general(
                xb, w1c, (((1,), (1,)), ((), ())),
                preferred_element_type=jnp.float32,
            ) + b1_ref[0, :, f0:f0 + f_chunk]
            h = jnp.maximum(h, 0.0)
            w2c = w2_ref[0, :, f0:f0 + f_chunk].astype(jnp.bfloat16)
            y = y + jax.lax.dot_general(
                h.astype(jnp.bfloat16), w2c, (((1,), (1,)), ((), ())),
                preferred_element_type=jnp.float32,
            )
        yg_ref[...] = y


def _final_kernel(x1_ref, y0_ref, y1_ref, w12_ref, g2_ref, bn2_ref, o_ref):
    w0 = w12_ref[:, 0:1]
    w1 = w12_ref[:, 1:2]
    z = x1_ref[...] + w0 * y0_ref[...] + w1 * y1_ref[...]
    o_ref[...] = _layer_norm(z, g2_ref[...], bn2_ref[...])


def _sc_dispatch(x1, s0, s1, n_slot):
    n_tok, d = x1.shape
    nw = 32
    chunk = n_tok // nw
    mesh = plsc.VectorSubcoreMesh(core_axis_name="c", subcore_axis_name="s")

    @functools.partial(
        pl.kernel, mesh=mesh,
        out_type=jax.ShapeDtypeStruct((n_slot, d), jnp.float32),
        scratch_types=[pltpu.VMEM((chunk,), jnp.int32),
                       pltpu.VMEM((chunk,), jnp.int32),
                       pltpu.VMEM((chunk, d), jnp.float32),
                       pltpu.SemaphoreType.DMA],
    )
    def disp(x1_hbm, s0_hbm, s1_hbm, xg_hbm, idx0_v, idx1_v, rows_v, sem):
        wid = lax.axis_index("s") * 2 + lax.axis_index("c")
        base = wid * chunk
        pltpu.sync_copy(x1_hbm.at[pl.ds(base, chunk)], rows_v)
        pltpu.sync_copy(s0_hbm.at[pl.ds(base, chunk)], idx0_v)
        pltpu.sync_copy(s1_hbm.at[pl.ds(base, chunk)], idx1_v)
        c0 = pltpu.async_copy(rows_v, xg_hbm.at[idx0_v], sem)
        c1 = pltpu.async_copy(rows_v, xg_hbm.at[idx1_v], sem)
        c0.wait()
        c1.wait()

    return disp(x1, s0, s1)


def _sc_combine(yg, s0, s1, n_tok):
    _, d = yg.shape
    nw = 32
    chunk = n_tok // nw
    mesh = plsc.VectorSubcoreMesh(core_axis_name="c", subcore_axis_name="s")

    @functools.partial(
        pl.kernel, mesh=mesh,
        out_type=(jax.ShapeDtypeStruct((n_tok, d), jnp.float32),
                  jax.ShapeDtypeStruct((n_tok, d), jnp.float32)),
        scratch_types=[pltpu.VMEM((chunk,), jnp.int32),
                       pltpu.VMEM((chunk,), jnp.int32),
                       pltpu.VMEM((chunk, d), jnp.float32),
                       pltpu.VMEM((chunk, d), jnp.float32),
                       pltpu.SemaphoreType.DMA],
    )
    def comb(yg_hbm, s0_hbm, s1_hbm, y0_hbm, y1_hbm, idx0_v, idx1_v, r0_v,
             r1_v, sem):
        wid = lax.axis_index("s") * 2 + lax.axis_index("c")
        base = wid * chunk
        pltpu.sync_copy(s0_hbm.at[pl.ds(base, chunk)], idx0_v)
        pltpu.sync_copy(s1_hbm.at[pl.ds(base, chunk)], idx1_v)
        c0 = pltpu.async_copy(yg_hbm.at[idx0_v], r0_v, sem)
        c1 = pltpu.async_copy(yg_hbm.at[idx1_v], r1_v, sem)
        c0.wait()
        c1.wait()
        pltpu.sync_copy(r0_v, y0_hbm.at[pl.ds(base, chunk)])
        pltpu.sync_copy(r1_v, y1_hbm.at[pl.ds(base, chunk)])

    return comb(yg, s0, s1)


def kernel(src, Wq, bq, Wk, bk, Wv, bv, Wo, bo, Wg, bg, W1e, b1e, W2e, b2e,
           g1, bn1, g2, bn2):
    Bq, T, D = src.shape
    E, F, _ = W1e.shape
    Hh = 12
    HD = D // Hh
    x = src.reshape(T, D)

    wcat = jnp.stack([Wq, Wk, Wv], axis=0)
    bcat = jnp.stack([bq, bk, bv], axis=0).reshape(3, 1, D)
    BM1 = 512
    q, kv = pl.pallas_call(
        _qkv_kernel,
        grid=(T // BM1, 3),
        in_specs=[
            pl.BlockSpec((BM1, D), lambda i, j: (i, 0)),
            pl.BlockSpec((1, D, D), lambda i, j: (j, 0, 0)),
            pl.BlockSpec((1, 1, D), lambda i, j: (j, 0, 0)),
        ],
        out_specs=[
            pl.BlockSpec((BM1, D), lambda i, j: (i, 0)),
            pl.BlockSpec((BM1, D), lambda i, j: (i, jax.lax.max(j - 1, 0))),
        ],
        out_shape=[
            jax.ShapeDtypeStruct((T, D), jnp.float32),
            jax.ShapeDtypeStruct((T, 2 * D), jnp.float32),
        ],
        compiler_params=pltpu.CompilerParams(
            dimension_semantics=("parallel", "arbitrary")),
    )(x, wcat, bcat)

    BMA = 512
    scale = float(HD) ** -0.5
    x1, i12, w12 = pl.pallas_call(
        functools.partial(_attn_kernel, scale=scale, n_heads=Hh, head_dim=HD),
        grid=(T // BMA,),
        in_specs=[
            pl.BlockSpec((BMA, D), lambda i: (i, 0)),
            pl.BlockSpec((T, 2 * D), lambda i: (0, 0)),
            pl.BlockSpec((BMA, D), lambda i: (i, 0)),
            pl.BlockSpec((D, D), lambda i: (0, 0)),
            pl.BlockSpec((1, D), lambda i: (0, 0)),
            pl.BlockSpec((1, D), lambda i: (0, 0)),
            pl.BlockSpec((1, D), lambda i: (0, 0)),
            pl.BlockSpec((E, D), lambda i: (0, 0)),
            pl.BlockSpec((1, E), lambda i: (0, 0)),
        ],
        out_specs=[
            pl.BlockSpec((BMA, D), lambda i: (i, 0)),
            pl.BlockSpec((BMA, 2), lambda i: (i, 0)),
            pl.BlockSpec((BMA, 2), lambda i: (i, 0)),
        ],
        out_shape=[
            jax.ShapeDtypeStruct((T, D), jnp.float32),
            jax.ShapeDtypeStruct((T, 2), jnp.int32),
            jax.ShapeDtypeStruct((T, 2), jnp.float32),
        ],
        compiler_params=pltpu.CompilerParams(
            dimension_semantics=("arbitrary",)),
    )(q, kv, x, Wo, bo.reshape(1, D), g1.reshape(1, D), bn1.reshape(1, D),
      Wg, bg.reshape(1, E))

    n_blk = 2 * T // _CAP + (E - 1)
    n_slot = n_blk * _CAP
    slot, be2 = pl.pallas_call(
        functools.partial(_route_kernel, n_exp=E, cap=_CAP, n_blk=n_blk,
                          row_chunk=512),
        grid=(1,),
        in_specs=[pl.BlockSpec((T, 2), lambda i: (0, 0))],
        out_specs=[
            pl.BlockSpec((T, 2), lambda i: (0, 0)),
            pl.BlockSpec((1, n_blk + 1), lambda i: (0, 0)),
        ],
        out_shape=[
            jax.ShapeDtypeStruct((T, 2), jnp.int32),
            jax.ShapeDtypeStruct((1, n_blk + 1), jnp.int32),
        ],
    )(i12)
    s0 = slot[:, 0]
    s1 = slot[:, 1]
    bex = be2.reshape(n_blk + 1)

    xg = _sc_dispatch(x1, s0, s1, n_slot)

    yg = pl.pallas_call(
        functools.partial(_ffn_kernel, f_chunk=512, n_blk=n_blk),
        grid_spec=pltpu.PrefetchScalarGridSpec(
            num_scalar_prefetch=1,
            grid=(n_blk,),
            in_specs=[
                pl.BlockSpec((_CAP, D), lambda b, be_r: (b, 0)),
                pl.BlockSpec((1, F, D), lambda b, be_r: (be_r[b], 0, 0)),
                pl.BlockSpec((1, 1, F), lambda b, be_r: (be_r[b], 0, 0)),
                pl.BlockSpec((1, D, F), lambda b, be_r: (be_r[b], 0, 0)),
                pl.BlockSpec((1, 1, D), lambda b, be_r: (be_r[b], 0, 0)),
            ],
            out_specs=pl.BlockSpec((_CAP, D), lambda b, be_r: (b, 0)),
        ),
        out_shape=jax.ShapeDtypeStruct((n_slot, D), jnp.float32),
        compiler_params=pltpu.CompilerParams(
            dimension_semantics=("arbitrary",)),
    )(bex, xg, W1e, b1e.reshape(E, 1, F), W2e, b2e.reshape(E, 1, D))

    y0, y1 = _sc_combine(yg, s0, s1, T)

    BM7 = 512
    out = pl.pallas_call(
        _final_kernel,
        grid=(T // BM7,),
        in_specs=[
            pl.BlockSpec((BM7, D), lambda i: (i, 0)),
            pl.BlockSpec((BM7, D), lambda i: (i, 0)),
            pl.BlockSpec((BM7, D), lambda i: (i, 0)),
            pl.BlockSpec((BM7, 2), lambda i: (i, 0)),
            pl.BlockSpec((1, D), lambda i: (0, 0)),
            pl.BlockSpec((1, D), lambda i: (0, 0)),
        ],
        out_specs=pl.BlockSpec((BM7, D), lambda i: (i, 0)),
        out_shape=jax.ShapeDtypeStruct((T, D), jnp.float32),
        compiler_params=pltpu.CompilerParams(
            dimension_semantics=("parallel",)),
    )(x1, y0, y1, w12, g2.reshape(1, D), bn2.reshape(1, D))

    return out.reshape(Bq, T, D)
